# baseline (speedup 1.0000x reference)
import functools
import jax
import jax.numpy as jnp
import numpy as np
from jax import lax
from jax.experimental import pallas as pl
from jax.experimental.pallas import tpu as pltpu
from jax.experimental.pallas import tpu_sc as plsc


def _celu(x):
    return jnp.where(x > 0, x, jnp.exp(x) - 1.0)


def _interleave_perm(h_dim, nl):
    perm = np.zeros(h_dim, dtype=np.int32)
    for g in range(h_dim // (2 * nl)):
        for m in range(nl):
            perm[2 * nl * g + 2 * m] = 2 * nl * g + m
            perm[2 * nl * g + 2 * m + 1] = 2 * nl * g + nl + m
    return perm


def _node_prep_body(hf_ref, cfp_ref, wnep_ref, bnep_ref, wcep_ref, bce_ref,
                    wcepp_ref, bcep_ref, hcbf_ref, c_ref):
    h_dim = wnep_ref.shape[1]
    h_p = _celu(
        jnp.dot(hf_ref[...], wnep_ref[...], preferred_element_type=jnp.float32)
        + bnep_ref[...])
    c_p = _celu(
        jnp.dot(cfp_ref[...], wcepp_ref[...], preferred_element_type=jnp.float32)
        + bcep_ref[...])
    hcbf_ref[:, :h_dim] = h_p.astype(jnp.bfloat16)
    hcbf_ref[:, h_dim:] = c_p.astype(jnp.bfloat16)
    c_ref[...] = _celu(
        jnp.dot(cfp_ref[...], wcep_ref[...], preferred_element_type=jnp.float32)
        + bce_ref[...])


def _node_prep(hfeats, cf_pad, W_ne_p, b_ne_p, W_ce_pad, b_ce, W_ce_pad_p,
               b_ce_p, n_blk):
    n = hfeats.shape[0]
    dn = hfeats.shape[1]
    h_dim = W_ne_p.shape[1]
    dcp = cf_pad.shape[1]
    grid = n // n_blk
    return pl.pallas_call(
        _node_prep_body,
        grid=(grid,),
        in_specs=[
            pl.BlockSpec((n_blk, dn), lambda i: (i, 0)),
            pl.BlockSpec((n_blk, dcp), lambda i: (i, 0)),
            pl.BlockSpec((dn, h_dim), lambda i: (0, 0)),
            pl.BlockSpec((1, h_dim), lambda i: (0, 0)),
            pl.BlockSpec((dcp, h_dim), lambda i: (0, 0)),
            pl.BlockSpec((1, h_dim), lambda i: (0, 0)),
            pl.BlockSpec((dcp, h_dim), lambda i: (0, 0)),
            pl.BlockSpec((1, h_dim), lambda i: (0, 0)),
        ],
        out_specs=[
            pl.BlockSpec((n_blk, 2 * h_dim), lambda i: (i, 0)),
            pl.BlockSpec((n_blk, h_dim), lambda i: (i, 0)),
        ],
        out_shape=[
            jax.ShapeDtypeStruct((n, 2 * h_dim), jnp.bfloat16),
            jax.ShapeDtypeStruct((n, h_dim), jnp.float32),
        ],
    )(hfeats, cf_pad, W_ne_p, b_ne_p.reshape(1, -1), W_ce_pad,
      b_ce.reshape(1, -1), W_ce_pad_p, b_ce_p.reshape(1, -1))


def _edge_prep_body(eft_ref, wee_ref, bee_ref, we1_ref, be1t_ref,
                    e_ref, efot_ref):
    eft = eft_ref[...]
    e = _celu(
        lax.dot_general(eft, wee_ref[...], (((0,), (0,)), ((), ())),
                        preferred_element_type=jnp.float32)
        + bee_ref[...])
    e_ref[...] = e
    efot_ref[...] = eft + _celu(
        lax.dot_general(we1_ref[...], e, (((0,), (1,)), ((), ())),
                        preferred_element_type=jnp.float32)
        + be1t_ref[...])


def _edge_prep(ef_t, W_ee, b_ee, W_e1, b_e1, e_blk):
    de, e_edges = ef_t.shape
    h_dim = W_ee.shape[1]
    grid = e_edges // e_blk
    return pl.pallas_call(
        _edge_prep_body,
        grid=(grid,),
        in_specs=[
            pl.BlockSpec((de, e_blk), lambda i: (0, i)),
            pl.BlockSpec((de, h_dim), lambda i: (0, 0)),
            pl.BlockSpec((1, h_dim), lambda i: (0, 0)),
            pl.BlockSpec((h_dim, de), lambda i: (0, 0)),
            pl.BlockSpec((de, 1), lambda i: (0, 0)),
        ],
        out_specs=[
            pl.BlockSpec((e_blk, h_dim), lambda i: (i, 0)),
            pl.BlockSpec((de, e_blk), lambda i: (0, i)),
        ],
        out_shape=[
            jax.ShapeDtypeStruct((e_edges, h_dim), jnp.float32),
            jax.ShapeDtypeStruct((de, e_edges), jnp.float32),
        ],
    )(ef_t, W_ee, b_ee.reshape(1, -1), W_e1, b_e1.reshape(-1, 1))


def _sc_edge_kernel(src, dst, e, hc, n_pad):
    e_edges, h_dim = e.shape
    hc_dim = hc.shape[1]
    nc, ns, nl = 2, 16, 16
    nw = nc * ns
    epw = e_edges // nw
    B = 40
    nb = epw // B
    npair = nb // 2
    rows_per_tile = n_pad // ns
    wchunk = 32
    nq = rows_per_tile // wchunk
    nvec = h_dim // nl
    ng = h_dim // (2 * nl)

    mesh = plsc.VectorSubcoreMesh(core_axis_name="c", subcore_axis_name="s")

    slot_types = [
        pltpu.VMEM((B,), jnp.int32),
        pltpu.VMEM((B,), jnp.int32),
        pltpu.VMEM((B, hc_dim), jnp.float32),
        pltpu.VMEM((B, hc_dim), jnp.float32),
        pltpu.VMEM((B, h_dim), jnp.float32),
        pltpu.VMEM((B, h_dim), jnp.float32),
        pltpu.SemaphoreType.DMA,
        pltpu.SemaphoreType.DMA,
    ]

    @functools.partial(
        pl.kernel,
        out_type=jax.ShapeDtypeStruct((nc, n_pad, h_dim), jnp.float32),
        mesh=mesh,
        scratch_types=slot_types + slot_types + [
            pltpu.VMEM((wchunk, h_dim), jnp.float32),
            pltpu.VMEM_SHARED((n_pad, h_dim), jnp.float32),
        ],
        compiler_params=pltpu.CompilerParams(needs_layout_passes=False),
    )
    def body(src_hbm, dst_hbm, e_hbm, hc_hbm, kout_hbm,
             srcv0, dstv0, hcv0, cdv0, ev0, xv0, gsem0, ssem0,
             srcv1, dstv1, hcv1, cdv1, ev1, xv1, gsem1, ssem1,
             zv, ksh):
        cid = lax.axis_index("c")
        sid = lax.axis_index("s")
        wid = sid * nc + cid
        base = wid * epw
        row0 = sid * rows_per_tile
        slots = ((srcv0, dstv0, hcv0, cdv0, ev0, xv0, gsem0, ssem0),
                 (srcv1, dstv1, hcv1, cdv1, ev1, xv1, gsem1, ssem1))

        def zrow(i, carry):
            for j in range(nvec):
                zv[i, pl.ds(j * nl, nl)] = jnp.zeros((nl,), jnp.float32)
            return carry
        lax.fori_loop(0, wchunk, zrow, 0)
        for q in range(nq):
            pltpu.sync_copy(zv, ksh.at[pl.ds(row0 + q * wchunk, wchunk)])
        plsc.subcore_barrier()

        def issue(t, sl):
            srcv, dstv, hcv, cdv, ev, xv, gsem, _ = sl
            off = base + t * B
            pltpu.sync_copy(src_hbm.at[pl.ds(off, B)], srcv)
            pltpu.sync_copy(dst_hbm.at[pl.ds(off, B)], dstv)
            pltpu.async_copy(e_hbm.at[pl.ds(off, B)], ev, gsem)
            pltpu.async_copy(hc_hbm.at[srcv], hcv, gsem)
            pltpu.async_copy(hc_hbm.at[dstv], cdv, gsem)

        def wait_gathers(sl):
            srcv, dstv, hcv, cdv, ev, xv, gsem, _ = sl
            pltpu.make_async_copy(e_hbm.at[pl.ds(0, B)], ev, gsem).wait()
            pltpu.make_async_copy(hc_hbm.at[srcv], hcv, gsem).wait()
            pltpu.make_async_copy(hc_hbm.at[dstv], cdv, gsem).wait()

        def compute_scatter(sl):
            srcv, dstv, hcv, cdv, ev, xv, _, ssem = sl

            for i in range(B):
                for g in range(ng):
                    sw = pl.ds(nl * g, nl)
                    swc = pl.ds(h_dim // 2 + nl * g, nl)
                    ha, hb = plsc.unpack(
                        plsc.bitcast(hcv[i, sw], jnp.bfloat16),
                        format=plsc.PackFormat.INTERLEAVED)
                    ca, cb = plsc.unpack(
                        plsc.bitcast(hcv[i, swc], jnp.bfloat16),
                        format=plsc.PackFormat.INTERLEAVED)
                    da, db = plsc.unpack(
                        plsc.bitcast(cdv[i, swc], jnp.bfloat16),
                        format=plsc.PackFormat.INTERLEAVED)
                    sa = pl.ds(2 * nl * g, nl)
                    sb = pl.ds(2 * nl * g + nl, nl)
                    xv[i, sa] = jnp.abs(da - ca) * (ha * ev[i, sa])
                    xv[i, sb] = jnp.abs(db - cb) * (hb * ev[i, sb])
            pltpu.async_copy(xv, ksh.at[dstv], ssem, add=True)

        def wait_scatter(sl):
            srcv, dstv, hcv, cdv, ev, xv, _, ssem = sl
            pltpu.make_async_copy(xv, ksh.at[dstv], ssem).wait()

        issue(0, slots[0])

        def pair(u, carry):
            t0 = 2 * u
            @pl.when(u > 0)
            def _():
                wait_scatter(slots[1])
            issue(t0 + 1, slots[1])
            wait_gathers(slots[0])
            compute_scatter(slots[0])
            @pl.when(u < npair - 1)
            def _():
                wait_scatter(slots[0])
                issue(t0 + 2, slots[0])
            wait_gathers(slots[1])
            compute_scatter(slots[1])
            return carry
        lax.fori_loop(0, npair, pair, 0)
        wait_scatter(slots[0])
        wait_scatter(slots[1])

        plsc.subcore_barrier()
        for q in range(nq):
            r = row0 + q * wchunk
            pltpu.sync_copy(ksh.at[pl.ds(r, wchunk)], zv)
            pltpu.sync_copy(zv, kout_hbm.at[cid, pl.ds(r, wchunk)])

    return body(src, dst, e, hc)


def _node_post_body(k0_ref, k1_ref, hf_ref, cfp_ref, c_ref,
                    wn1_ref, bn1_ref, wn2_ref, bn2_ref, wc1p_ref, bc1p_ref,
                    hfo_ref, cfo_ref):
    k = k0_ref[...] + k1_ref[...]
    t = _celu(
        jnp.dot(k, wn1_ref[...], preferred_element_type=jnp.float32)
        + bn1_ref[...])
    hfo_ref[...] = hf_ref[...] + _celu(
        jnp.dot(t, wn2_ref[...], preferred_element_type=jnp.float32)
        + bn2_ref[...])
    cfo_ref[...] = cfp_ref[...] + _celu(
        jnp.dot(c_ref[...], wc1p_ref[...], preferred_element_type=jnp.float32)
        + bc1p_ref[...])


def _node_post(k0, k1, hfeats, cf_pad, c, W_n1, b_n1, W_n2, b_n2,
               W_c1_pad, b_c1_pad, n_blk):
    n, h_dim = k0.shape
    dn = hfeats.shape[1]
    dcp = cf_pad.shape[1]
    grid = n // n_blk
    return pl.pallas_call(
        _node_post_body,
        grid=(grid,),
        in_specs=[
            pl.BlockSpec((n_blk, h_dim), lambda i: (i, 0)),
            pl.BlockSpec((n_blk, h_dim), lambda i: (i, 0)),
            pl.BlockSpec((n_blk, dn), lambda i: (i, 0)),
            pl.BlockSpec((n_blk, dcp), lambda i: (i, 0)),
            pl.BlockSpec((n_blk, h_dim), lambda i: (i, 0)),
            pl.BlockSpec((h_dim, h_dim), lambda i: (0, 0)),
            pl.BlockSpec((1, h_dim), lambda i: (0, 0)),
            pl.BlockSpec((h_dim, dn), lambda i: (0, 0)),
            pl.BlockSpec((1, dn), lambda i: (0, 0)),
            pl.BlockSpec((h_dim, dcp), lambda i: (0, 0)),
            pl.BlockSpec((1, dcp), lambda i: (0, 0)),
        ],
        out_specs=[
            pl.BlockSpec((n_blk, dn), lambda i: (i, 0)),
            pl.BlockSpec((n_blk, dcp), lambda i: (i, 0)),
        ],
        out_shape=[
            jax.ShapeDtypeStruct((n, dn), jnp.float32),
            jax.ShapeDtypeStruct((n, dcp), jnp.float32),
        ],
    )(k0, k1, hfeats, cf_pad, c, W_n1, b_n1.reshape(1, -1), W_n2,
      b_n2.reshape(1, -1), W_c1_pad, b_c1_pad.reshape(1, -1))


def kernel(hfeats, cfeats, efeats, edge_index, W_ne, b_ne, W_ee, b_ee, W_ce,
           b_ce, W_n1, b_n1, W_n2, b_n2, W_e1, b_e1, W_c1, b_c1):
    n = hfeats.shape[0]
    dc = cfeats.shape[1]
    dcp = 8
    h_dim = W_ne.shape[1]

    src = edge_index[0].astype(jnp.int32)
    dst = edge_index[1].astype(jnp.int32)

    cf_pad = jnp.pad(cfeats, ((0, 0), (0, dcp - dc)))
    W_ce_pad = jnp.pad(W_ce, ((0, dcp - dc), (0, 0)))
    W_c1_pad = jnp.pad(W_c1, ((0, 0), (0, dcp - dc)))
    b_c1_pad = jnp.pad(b_c1, (0, dcp - dc))

    perm = jnp.asarray(_interleave_perm(h_dim, 16))
    W_ne_p = W_ne[:, perm]
    b_ne_p = b_ne[perm]
    W_ce_pad_p = W_ce_pad[:, perm]
    b_ce_p = b_ce[perm]

    hc_bf, c = _node_prep(hfeats, cf_pad, W_ne_p, b_ne_p, W_ce_pad,
                          b_ce, W_ce_pad_p, b_ce_p, n_blk=1000)
    hc_packed = lax.bitcast_convert_type(
        hc_bf.reshape(n, h_dim, 2), jnp.float32)
    e, efo_t = _edge_prep(efeats.T, W_ee, b_ee, W_e1, b_e1, e_blk=2560)
    efeats_out = efo_t.T

    n_pad = 10240
    k_parts = _sc_edge_kernel(src, dst, e, hc_packed, n_pad)

    hfeats_out, cf_out_pad = _node_post(
        k_parts[0, :n], k_parts[1, :n], hfeats, cf_pad, c,
        W_n1, b_n1, W_n2, b_n2, W_c1_pad, b_c1_pad, n_blk=1000)
    cfeats_out = cf_out_pad[:, :dc]
    return (hfeats_out, cfeats_out, efeats_out)


# speedup vs baseline: 1.2602x; 1.2602x over previous
import functools
import jax
import jax.numpy as jnp
import numpy as np
from jax import lax
from jax.experimental import pallas as pl
from jax.experimental.pallas import tpu as pltpu
from jax.experimental.pallas import tpu_sc as plsc


def _celu(x):
    return jnp.where(x > 0, x, jnp.exp(x) - 1.0)


def _interleave_perm(h_dim, nl):
    perm = np.zeros(h_dim, dtype=np.int32)
    for g in range(h_dim // (2 * nl)):
        for m in range(nl):
            perm[2 * nl * g + 2 * m] = 2 * nl * g + m
            perm[2 * nl * g + 2 * m + 1] = 2 * nl * g + nl + m
    return perm


def _node_prep_body(hf_ref, cfp_ref, wnep_ref, bnep_ref, wcep_ref, bce_ref,
                    wcepp_ref, bcep_ref, hcbf_ref, c_ref):
    h_dim = wnep_ref.shape[1]
    h_p = _celu(
        jnp.dot(hf_ref[...], wnep_ref[...], preferred_element_type=jnp.float32)
        + bnep_ref[...])
    c_p = _celu(
        jnp.dot(cfp_ref[...], wcepp_ref[...], preferred_element_type=jnp.float32)
        + bcep_ref[...])
    hcbf_ref[:, :h_dim] = h_p.astype(jnp.bfloat16)
    hcbf_ref[:, h_dim:] = c_p.astype(jnp.bfloat16)
    c_ref[...] = _celu(
        jnp.dot(cfp_ref[...], wcep_ref[...], preferred_element_type=jnp.float32)
        + bce_ref[...])


def _node_prep(hfeats, cf_pad, W_ne_p, b_ne_p, W_ce_pad, b_ce, W_ce_pad_p,
               b_ce_p, n_blk):
    n = hfeats.shape[0]
    dn = hfeats.shape[1]
    h_dim = W_ne_p.shape[1]
    dcp = cf_pad.shape[1]
    grid = n // n_blk
    return pl.pallas_call(
        _node_prep_body,
        grid=(grid,),
        in_specs=[
            pl.BlockSpec((n_blk, dn), lambda i: (i, 0)),
            pl.BlockSpec((n_blk, dcp), lambda i: (i, 0)),
            pl.BlockSpec((dn, h_dim), lambda i: (0, 0)),
            pl.BlockSpec((1, h_dim), lambda i: (0, 0)),
            pl.BlockSpec((dcp, h_dim), lambda i: (0, 0)),
            pl.BlockSpec((1, h_dim), lambda i: (0, 0)),
            pl.BlockSpec((dcp, h_dim), lambda i: (0, 0)),
            pl.BlockSpec((1, h_dim), lambda i: (0, 0)),
        ],
        out_specs=[
            pl.BlockSpec((n_blk, 2 * h_dim), lambda i: (i, 0)),
            pl.BlockSpec((n_blk, h_dim), lambda i: (i, 0)),
        ],
        out_shape=[
            jax.ShapeDtypeStruct((n, 2 * h_dim), jnp.bfloat16),
            jax.ShapeDtypeStruct((n, h_dim), jnp.float32),
        ],
    )(hfeats, cf_pad, W_ne_p, b_ne_p.reshape(1, -1), W_ce_pad,
      b_ce.reshape(1, -1), W_ce_pad_p, b_ce_p.reshape(1, -1))


def _edge_prep_body(eft_ref, wee_ref, bee_ref, we1_ref, be1t_ref,
                    e_ref, efot_ref):
    eft = eft_ref[...]
    e = _celu(
        lax.dot_general(eft, wee_ref[...], (((0,), (0,)), ((), ())),
                        preferred_element_type=jnp.float32)
        + bee_ref[...])
    e_ref[...] = e
    efot_ref[...] = eft + _celu(
        lax.dot_general(we1_ref[...], e, (((0,), (1,)), ((), ())),
                        preferred_element_type=jnp.float32)
        + be1t_ref[...])


def _edge_prep(ef_t, W_ee, b_ee, W_e1, b_e1, e_blk):
    de, e_edges = ef_t.shape
    h_dim = W_ee.shape[1]
    grid = e_edges // e_blk
    return pl.pallas_call(
        _edge_prep_body,
        grid=(grid,),
        in_specs=[
            pl.BlockSpec((de, e_blk), lambda i: (0, i)),
            pl.BlockSpec((de, h_dim), lambda i: (0, 0)),
            pl.BlockSpec((1, h_dim), lambda i: (0, 0)),
            pl.BlockSpec((h_dim, de), lambda i: (0, 0)),
            pl.BlockSpec((de, 1), lambda i: (0, 0)),
        ],
        out_specs=[
            pl.BlockSpec((e_blk, h_dim), lambda i: (i, 0)),
            pl.BlockSpec((de, e_blk), lambda i: (0, i)),
        ],
        out_shape=[
            jax.ShapeDtypeStruct((e_edges, h_dim), jnp.float32),
            jax.ShapeDtypeStruct((de, e_edges), jnp.float32),
        ],
    )(ef_t, W_ee, b_ee.reshape(1, -1), W_e1, b_e1.reshape(-1, 1))


def _sc_edge_kernel(src, dst, e, hc, n_pad):
    e_edges, h_dim = e.shape
    hc_dim = hc.shape[1]
    nc, ns, nl = 2, 16, 16
    nw = nc * ns
    epw = e_edges // nw
    B = 40
    nb = epw // B
    npair = nb // 2
    rows_per_tile = n_pad // ns
    wchunk = 32
    nq = rows_per_tile // wchunk
    nvec = h_dim // nl
    ng = h_dim // (2 * nl)

    mesh = plsc.VectorSubcoreMesh(core_axis_name="c", subcore_axis_name="s")

    slot_types = [
        pltpu.VMEM((B,), jnp.int32),
        pltpu.VMEM((B,), jnp.int32),
        pltpu.VMEM((B, hc_dim), jnp.float32),
        pltpu.VMEM((B, hc_dim), jnp.float32),
        pltpu.VMEM((B, h_dim), jnp.float32),
        pltpu.VMEM((B, h_dim), jnp.float32),
        pltpu.SemaphoreType.DMA,
        pltpu.SemaphoreType.DMA,
    ]

    @functools.partial(
        pl.kernel,
        out_type=jax.ShapeDtypeStruct((nc, n_pad, h_dim), jnp.float32),
        mesh=mesh,
        scratch_types=slot_types + slot_types + [
            pltpu.VMEM((wchunk, h_dim), jnp.float32),
            pltpu.VMEM_SHARED((n_pad, h_dim), jnp.float32),
        ],
        compiler_params=pltpu.CompilerParams(needs_layout_passes=False),
    )
    def body(src_hbm, dst_hbm, e_hbm, hc_hbm, kout_hbm,
             srcv0, dstv0, hcv0, cdv0, ev0, xv0, gsem0, ssem0,
             srcv1, dstv1, hcv1, cdv1, ev1, xv1, gsem1, ssem1,
             zv, ksh):
        cid = lax.axis_index("c")
        sid = lax.axis_index("s")
        wid = sid * nc + cid
        base = wid * epw
        row0 = sid * rows_per_tile
        slots = ((srcv0, dstv0, hcv0, cdv0, ev0, xv0, gsem0, ssem0),
                 (srcv1, dstv1, hcv1, cdv1, ev1, xv1, gsem1, ssem1))

        def zrow(i, carry):
            for j in range(nvec):
                zv[i, pl.ds(j * nl, nl)] = jnp.zeros((nl,), jnp.float32)
            return carry
        lax.fori_loop(0, wchunk, zrow, 0)
        for q in range(nq):
            pltpu.sync_copy(zv, ksh.at[pl.ds(row0 + q * wchunk, wchunk)])
        plsc.subcore_barrier()

        def issue(t, sl):
            srcv, dstv, hcv, cdv, ev, xv, gsem, _ = sl
            off = base + t * B
            pltpu.sync_copy(src_hbm.at[pl.ds(off, B)], srcv)
            pltpu.sync_copy(dst_hbm.at[pl.ds(off, B)], dstv)
            pltpu.async_copy(e_hbm.at[pl.ds(off, B)], ev, gsem)
            pltpu.async_copy(hc_hbm.at[srcv], hcv, gsem)
            pltpu.async_copy(hc_hbm.at[dstv], cdv, gsem)

        def wait_gathers(sl):
            srcv, dstv, hcv, cdv, ev, xv, gsem, _ = sl
            pltpu.make_async_copy(e_hbm.at[pl.ds(0, B)], ev, gsem).wait()
            pltpu.make_async_copy(hc_hbm.at[srcv], hcv, gsem).wait()
            pltpu.make_async_copy(hc_hbm.at[dstv], cdv, gsem).wait()

        def compute_scatter(sl):
            srcv, dstv, hcv, cdv, ev, xv, _, ssem = sl

            @plsc.parallel_loop(0, B, unroll=4)
            def crow(i):
                for g in range(ng):
                    sw = pl.ds(nl * g, nl)
                    swc = pl.ds(h_dim // 2 + nl * g, nl)
                    ha, hb = plsc.unpack(
                        plsc.bitcast(hcv[i, sw], jnp.bfloat16),
                        format=plsc.PackFormat.INTERLEAVED)
                    ca, cb = plsc.unpack(
                        plsc.bitcast(hcv[i, swc], jnp.bfloat16),
                        format=plsc.PackFormat.INTERLEAVED)
                    da, db = plsc.unpack(
                        plsc.bitcast(cdv[i, swc], jnp.bfloat16),
                        format=plsc.PackFormat.INTERLEAVED)
                    sa = pl.ds(2 * nl * g, nl)
                    sb = pl.ds(2 * nl * g + nl, nl)
                    xv[i, sa] = jnp.abs(da - ca) * (ha * ev[i, sa])
                    xv[i, sb] = jnp.abs(db - cb) * (hb * ev[i, sb])
            pltpu.async_copy(xv, ksh.at[dstv], ssem, add=True)

        def wait_scatter(sl):
            srcv, dstv, hcv, cdv, ev, xv, _, ssem = sl
            pltpu.make_async_copy(xv, ksh.at[dstv], ssem).wait()

        issue(0, slots[0])

        def pair(u, carry):
            t0 = 2 * u
            @pl.when(u > 0)
            def _():
                wait_scatter(slots[1])
            issue(t0 + 1, slots[1])
            wait_gathers(slots[0])
            compute_scatter(slots[0])
            @pl.when(u < npair - 1)
            def _():
                wait_scatter(slots[0])
                issue(t0 + 2, slots[0])
            wait_gathers(slots[1])
            compute_scatter(slots[1])
            return carry
        lax.fori_loop(0, npair, pair, 0)
        wait_scatter(slots[0])
        wait_scatter(slots[1])

        plsc.subcore_barrier()
        for q in range(nq):
            r = row0 + q * wchunk
            pltpu.sync_copy(ksh.at[pl.ds(r, wchunk)], zv)
            pltpu.sync_copy(zv, kout_hbm.at[cid, pl.ds(r, wchunk)])

    return body(src, dst, e, hc)


def _node_post_body(k0_ref, k1_ref, hf_ref, cfp_ref, c_ref,
                    wn1_ref, bn1_ref, wn2_ref, bn2_ref, wc1p_ref, bc1p_ref,
                    hfo_ref, cfo_ref):
    k = k0_ref[...] + k1_ref[...]
    t = _celu(
        jnp.dot(k, wn1_ref[...], preferred_element_type=jnp.float32)
        + bn1_ref[...])
    hfo_ref[...] = hf_ref[...] + _celu(
        jnp.dot(t, wn2_ref[...], preferred_element_type=jnp.float32)
        + bn2_ref[...])
    cfo_ref[...] = cfp_ref[...] + _celu(
        jnp.dot(c_ref[...], wc1p_ref[...], preferred_element_type=jnp.float32)
        + bc1p_ref[...])


def _node_post(k0, k1, hfeats, cf_pad, c, W_n1, b_n1, W_n2, b_n2,
               W_c1_pad, b_c1_pad, n_blk):
    n, h_dim = k0.shape
    dn = hfeats.shape[1]
    dcp = cf_pad.shape[1]
    grid = n // n_blk
    return pl.pallas_call(
        _node_post_body,
        grid=(grid,),
        in_specs=[
            pl.BlockSpec((n_blk, h_dim), lambda i: (i, 0)),
            pl.BlockSpec((n_blk, h_dim), lambda i: (i, 0)),
            pl.BlockSpec((n_blk, dn), lambda i: (i, 0)),
            pl.BlockSpec((n_blk, dcp), lambda i: (i, 0)),
            pl.BlockSpec((n_blk, h_dim), lambda i: (i, 0)),
            pl.BlockSpec((h_dim, h_dim), lambda i: (0, 0)),
            pl.BlockSpec((1, h_dim), lambda i: (0, 0)),
            pl.BlockSpec((h_dim, dn), lambda i: (0, 0)),
            pl.BlockSpec((1, dn), lambda i: (0, 0)),
            pl.BlockSpec((h_dim, dcp), lambda i: (0, 0)),
            pl.BlockSpec((1, dcp), lambda i: (0, 0)),
        ],
        out_specs=[
            pl.BlockSpec((n_blk, dn), lambda i: (i, 0)),
            pl.BlockSpec((n_blk, dcp), lambda i: (i, 0)),
        ],
        out_shape=[
            jax.ShapeDtypeStruct((n, dn), jnp.float32),
            jax.ShapeDtypeStruct((n, dcp), jnp.float32),
        ],
    )(k0, k1, hfeats, cf_pad, c, W_n1, b_n1.reshape(1, -1), W_n2,
      b_n2.reshape(1, -1), W_c1_pad, b_c1_pad.reshape(1, -1))


def kernel(hfeats, cfeats, efeats, edge_index, W_ne, b_ne, W_ee, b_ee, W_ce,
           b_ce, W_n1, b_n1, W_n2, b_n2, W_e1, b_e1, W_c1, b_c1):
    n = hfeats.shape[0]
    dc = cfeats.shape[1]
    dcp = 8
    h_dim = W_ne.shape[1]

    src = edge_index[0].astype(jnp.int32)
    dst = edge_index[1].astype(jnp.int32)

    cf_pad = jnp.pad(cfeats, ((0, 0), (0, dcp - dc)))
    W_ce_pad = jnp.pad(W_ce, ((0, dcp - dc), (0, 0)))
    W_c1_pad = jnp.pad(W_c1, ((0, 0), (0, dcp - dc)))
    b_c1_pad = jnp.pad(b_c1, (0, dcp - dc))

    perm = jnp.asarray(_interleave_perm(h_dim, 16))
    W_ne_p = W_ne[:, perm]
    b_ne_p = b_ne[perm]
    W_ce_pad_p = W_ce_pad[:, perm]
    b_ce_p = b_ce[perm]

    hc_bf, c = _node_prep(hfeats, cf_pad, W_ne_p, b_ne_p, W_ce_pad,
                          b_ce, W_ce_pad_p, b_ce_p, n_blk=1000)
    hc_packed = lax.bitcast_convert_type(
        hc_bf.reshape(n, h_dim, 2), jnp.float32)
    e, efo_t = _edge_prep(efeats.T, W_ee, b_ee, W_e1, b_e1, e_blk=2560)
    efeats_out = efo_t.T

    n_pad = 10240
    k_parts = _sc_edge_kernel(src, dst, e, hc_packed, n_pad)

    hfeats_out, cf_out_pad = _node_post(
        k_parts[0, :n], k_parts[1, :n], hfeats, cf_pad, c,
        W_n1, b_n1, W_n2, b_n2, W_c1_pad, b_c1_pad, n_blk=1000)
    cfeats_out = cf_out_pad[:, :dc]
    return (hfeats_out, cfeats_out, efeats_out)


# speedup vs baseline: 1.3352x; 1.0596x over previous
import functools
import jax
import jax.numpy as jnp
import numpy as np
from jax import lax
from jax.experimental import pallas as pl
from jax.experimental.pallas import tpu as pltpu
from jax.experimental.pallas import tpu_sc as plsc


def _celu(x):
    return jnp.where(x > 0, x, jnp.exp(x) - 1.0)


def _interleave_perm(h_dim, nl):
    perm = np.zeros(h_dim, dtype=np.int32)
    for g in range(h_dim // (2 * nl)):
        for m in range(nl):
            perm[2 * nl * g + 2 * m] = 2 * nl * g + m
            perm[2 * nl * g + 2 * m + 1] = 2 * nl * g + nl + m
    return perm


def _node_prep_body(hf_ref, cfp_ref, wnep_ref, bnep_ref, wcep_ref, bce_ref,
                    wcepp_ref, bcep_ref, hcbf_ref, c_ref):
    h_dim = wnep_ref.shape[1]
    h_p = _celu(
        jnp.dot(hf_ref[...], wnep_ref[...], preferred_element_type=jnp.float32)
        + bnep_ref[...])
    c_p = _celu(
        jnp.dot(cfp_ref[...], wcepp_ref[...], preferred_element_type=jnp.float32)
        + bcep_ref[...])
    hcbf_ref[:, :h_dim] = h_p.astype(jnp.bfloat16)
    hcbf_ref[:, h_dim:] = c_p.astype(jnp.bfloat16)
    c_ref[...] = _celu(
        jnp.dot(cfp_ref[...], wcep_ref[...], preferred_element_type=jnp.float32)
        + bce_ref[...])


def _node_prep(hfeats, cf_pad, W_ne_p, b_ne_p, W_ce_pad, b_ce, W_ce_pad_p,
               b_ce_p, n_blk):
    n = hfeats.shape[0]
    dn = hfeats.shape[1]
    h_dim = W_ne_p.shape[1]
    dcp = cf_pad.shape[1]
    grid = n // n_blk
    return pl.pallas_call(
        _node_prep_body,
        grid=(grid,),
        in_specs=[
            pl.BlockSpec((n_blk, dn), lambda i: (i, 0)),
            pl.BlockSpec((n_blk, dcp), lambda i: (i, 0)),
            pl.BlockSpec((dn, h_dim), lambda i: (0, 0)),
            pl.BlockSpec((1, h_dim), lambda i: (0, 0)),
            pl.BlockSpec((dcp, h_dim), lambda i: (0, 0)),
            pl.BlockSpec((1, h_dim), lambda i: (0, 0)),
            pl.BlockSpec((dcp, h_dim), lambda i: (0, 0)),
            pl.BlockSpec((1, h_dim), lambda i: (0, 0)),
        ],
        out_specs=[
            pl.BlockSpec((n_blk, 2 * h_dim), lambda i: (i, 0)),
            pl.BlockSpec((n_blk, h_dim), lambda i: (i, 0)),
        ],
        out_shape=[
            jax.ShapeDtypeStruct((n, 2 * h_dim), jnp.bfloat16),
            jax.ShapeDtypeStruct((n, h_dim), jnp.float32),
        ],
    )(hfeats, cf_pad, W_ne_p, b_ne_p.reshape(1, -1), W_ce_pad,
      b_ce.reshape(1, -1), W_ce_pad_p, b_ce_p.reshape(1, -1))


def _edge_prep_body(eft_ref, wee_ref, bee_ref, we1_ref, be1t_ref,
                    e_ref, efot_ref):
    eft = eft_ref[...]
    e = _celu(
        lax.dot_general(eft, wee_ref[...], (((0,), (0,)), ((), ())),
                        preferred_element_type=jnp.float32)
        + bee_ref[...])
    e_ref[...] = e
    efot_ref[...] = eft + _celu(
        lax.dot_general(we1_ref[...], e, (((0,), (1,)), ((), ())),
                        preferred_element_type=jnp.float32)
        + be1t_ref[...])


def _edge_prep(ef_t, W_ee, b_ee, W_e1, b_e1, e_blk, off_blk, n_blk):
    de = ef_t.shape[0]
    h_dim = W_ee.shape[1]
    n_edges = n_blk * e_blk
    return pl.pallas_call(
        _edge_prep_body,
        grid=(n_blk,),
        in_specs=[
            pl.BlockSpec((de, e_blk), lambda i: (0, i + off_blk)),
            pl.BlockSpec((de, h_dim), lambda i: (0, 0)),
            pl.BlockSpec((1, h_dim), lambda i: (0, 0)),
            pl.BlockSpec((h_dim, de), lambda i: (0, 0)),
            pl.BlockSpec((de, 1), lambda i: (0, 0)),
        ],
        out_specs=[
            pl.BlockSpec((e_blk, h_dim), lambda i: (i, 0)),
            pl.BlockSpec((de, e_blk), lambda i: (0, i)),
        ],
        out_shape=[
            jax.ShapeDtypeStruct((n_edges, h_dim), jnp.float32),
            jax.ShapeDtypeStruct((de, n_edges), jnp.float32),
        ],
    )(ef_t, W_ee, b_ee.reshape(1, -1), W_e1, b_e1.reshape(-1, 1))


def _sc_edge_kernel(src, dst, e, hc, n_pad):
    e_edges, h_dim = e.shape
    hc_dim = hc.shape[1]
    nc, ns, nl = 2, 16, 16
    nw = nc * ns
    epw = e_edges // nw
    B = 40
    nb = epw // B
    npair = nb // 2
    rows_per_tile = n_pad // ns
    wchunk = 32
    nq = rows_per_tile // wchunk
    nvec = h_dim // nl
    ng = h_dim // (2 * nl)

    mesh = plsc.VectorSubcoreMesh(core_axis_name="c", subcore_axis_name="s")

    slot_types = [
        pltpu.VMEM((B,), jnp.int32),
        pltpu.VMEM((B,), jnp.int32),
        pltpu.VMEM((B, hc_dim), jnp.float32),
        pltpu.VMEM((B, hc_dim), jnp.float32),
        pltpu.VMEM((B, h_dim), jnp.float32),
        pltpu.VMEM((B, h_dim), jnp.float32),
        pltpu.SemaphoreType.DMA,
        pltpu.SemaphoreType.DMA,
    ]

    @functools.partial(
        pl.kernel,
        out_type=jax.ShapeDtypeStruct((nc, n_pad, h_dim), jnp.float32),
        mesh=mesh,
        scratch_types=slot_types + slot_types + [
            pltpu.VMEM((wchunk, h_dim), jnp.float32),
            pltpu.VMEM_SHARED((n_pad, h_dim), jnp.float32),
        ],
        compiler_params=pltpu.CompilerParams(needs_layout_passes=False),
    )
    def body(src_hbm, dst_hbm, e_hbm, hc_hbm, kout_hbm,
             srcv0, dstv0, hcv0, cdv0, ev0, xv0, gsem0, ssem0,
             srcv1, dstv1, hcv1, cdv1, ev1, xv1, gsem1, ssem1,
             zv, ksh):
        cid = lax.axis_index("c")
        sid = lax.axis_index("s")
        wid = sid * nc + cid
        base = wid * epw
        row0 = sid * rows_per_tile
        slots = ((srcv0, dstv0, hcv0, cdv0, ev0, xv0, gsem0, ssem0),
                 (srcv1, dstv1, hcv1, cdv1, ev1, xv1, gsem1, ssem1))

        def zrow(i, carry):
            for j in range(nvec):
                zv[i, pl.ds(j * nl, nl)] = jnp.zeros((nl,), jnp.float32)
            return carry
        lax.fori_loop(0, wchunk, zrow, 0)
        for q in range(nq):
            pltpu.sync_copy(zv, ksh.at[pl.ds(row0 + q * wchunk, wchunk)])
        plsc.subcore_barrier()

        def issue(t, sl):
            srcv, dstv, hcv, cdv, ev, xv, gsem, _ = sl
            off = base + t * B
            pltpu.sync_copy(src_hbm.at[pl.ds(off, B)], srcv)
            pltpu.sync_copy(dst_hbm.at[pl.ds(off, B)], dstv)
            pltpu.async_copy(e_hbm.at[pl.ds(off, B)], ev, gsem)
            pltpu.async_copy(hc_hbm.at[srcv], hcv, gsem)
            pltpu.async_copy(hc_hbm.at[dstv], cdv, gsem)

        def wait_gathers(sl):
            srcv, dstv, hcv, cdv, ev, xv, gsem, _ = sl
            pltpu.make_async_copy(e_hbm.at[pl.ds(0, B)], ev, gsem).wait()
            pltpu.make_async_copy(hc_hbm.at[srcv], hcv, gsem).wait()
            pltpu.make_async_copy(hc_hbm.at[dstv], cdv, gsem).wait()

        def compute_scatter(sl):
            srcv, dstv, hcv, cdv, ev, xv, _, ssem = sl

            @plsc.parallel_loop(0, B, unroll=4)
            def crow(i):
                for g in range(ng):
                    sw = pl.ds(nl * g, nl)
                    swc = pl.ds(h_dim // 2 + nl * g, nl)
                    ha, hb = plsc.unpack(
                        plsc.bitcast(hcv[i, sw], jnp.bfloat16),
                        format=plsc.PackFormat.INTERLEAVED)
                    ca, cb = plsc.unpack(
                        plsc.bitcast(hcv[i, swc], jnp.bfloat16),
                        format=plsc.PackFormat.INTERLEAVED)
                    da, db = plsc.unpack(
                        plsc.bitcast(cdv[i, swc], jnp.bfloat16),
                        format=plsc.PackFormat.INTERLEAVED)
                    sa = pl.ds(2 * nl * g, nl)
                    sb = pl.ds(2 * nl * g + nl, nl)
                    xv[i, sa] = jnp.abs(da - ca) * (ha * ev[i, sa])
                    xv[i, sb] = jnp.abs(db - cb) * (hb * ev[i, sb])
            pltpu.async_copy(xv, ksh.at[dstv], ssem, add=True)

        def wait_scatter(sl):
            srcv, dstv, hcv, cdv, ev, xv, _, ssem = sl
            pltpu.make_async_copy(xv, ksh.at[dstv], ssem).wait()

        issue(0, slots[0])

        def pair(u, carry):
            t0 = 2 * u
            @pl.when(u > 0)
            def _():
                wait_scatter(slots[1])
            issue(t0 + 1, slots[1])
            wait_gathers(slots[0])
            compute_scatter(slots[0])
            @pl.when(u < npair - 1)
            def _():
                wait_scatter(slots[0])
                issue(t0 + 2, slots[0])
            wait_gathers(slots[1])
            compute_scatter(slots[1])
            return carry
        lax.fori_loop(0, npair, pair, 0)
        wait_scatter(slots[0])
        wait_scatter(slots[1])

        plsc.subcore_barrier()
        for q in range(nq):
            r = row0 + q * wchunk
            pltpu.sync_copy(ksh.at[pl.ds(r, wchunk)], zv)
            pltpu.sync_copy(zv, kout_hbm.at[cid, pl.ds(r, wchunk)])

    return body(src, dst, e, hc)


def _node_post_body(k0_ref, k1_ref, k2_ref, k3_ref, hf_ref, cfp_ref, c_ref,
                    wn1_ref, bn1_ref, wn2_ref, bn2_ref, wc1p_ref, bc1p_ref,
                    hfo_ref, cfo_ref):
    k = (k0_ref[...] + k1_ref[...]) + (k2_ref[...] + k3_ref[...])
    t = _celu(
        jnp.dot(k, wn1_ref[...], preferred_element_type=jnp.float32)
        + bn1_ref[...])
    hfo_ref[...] = hf_ref[...] + _celu(
        jnp.dot(t, wn2_ref[...], preferred_element_type=jnp.float32)
        + bn2_ref[...])
    cfo_ref[...] = cfp_ref[...] + _celu(
        jnp.dot(c_ref[...], wc1p_ref[...], preferred_element_type=jnp.float32)
        + bc1p_ref[...])


def _node_post(k0, k1, k2, k3, hfeats, cf_pad, c, W_n1, b_n1, W_n2, b_n2,
               W_c1_pad, b_c1_pad, n_blk):
    n, h_dim = k0.shape
    dn = hfeats.shape[1]
    dcp = cf_pad.shape[1]
    grid = n // n_blk
    return pl.pallas_call(
        _node_post_body,
        grid=(grid,),
        in_specs=[
            pl.BlockSpec((n_blk, h_dim), lambda i: (i, 0)),
            pl.BlockSpec((n_blk, h_dim), lambda i: (i, 0)),
            pl.BlockSpec((n_blk, h_dim), lambda i: (i, 0)),
            pl.BlockSpec((n_blk, h_dim), lambda i: (i, 0)),
            pl.BlockSpec((n_blk, dn), lambda i: (i, 0)),
            pl.BlockSpec((n_blk, dcp), lambda i: (i, 0)),
            pl.BlockSpec((n_blk, h_dim), lambda i: (i, 0)),
            pl.BlockSpec((h_dim, h_dim), lambda i: (0, 0)),
            pl.BlockSpec((1, h_dim), lambda i: (0, 0)),
            pl.BlockSpec((h_dim, dn), lambda i: (0, 0)),
            pl.BlockSpec((1, dn), lambda i: (0, 0)),
            pl.BlockSpec((h_dim, dcp), lambda i: (0, 0)),
            pl.BlockSpec((1, dcp), lambda i: (0, 0)),
        ],
        out_specs=[
            pl.BlockSpec((n_blk, dn), lambda i: (i, 0)),
            pl.BlockSpec((n_blk, dcp), lambda i: (i, 0)),
        ],
        out_shape=[
            jax.ShapeDtypeStruct((n, dn), jnp.float32),
            jax.ShapeDtypeStruct((n, dcp), jnp.float32),
        ],
    )(k0, k1, k2, k3, hfeats, cf_pad, c, W_n1, b_n1.reshape(1, -1), W_n2,
      b_n2.reshape(1, -1), W_c1_pad, b_c1_pad.reshape(1, -1))


def kernel(hfeats, cfeats, efeats, edge_index, W_ne, b_ne, W_ee, b_ee, W_ce,
           b_ce, W_n1, b_n1, W_n2, b_n2, W_e1, b_e1, W_c1, b_c1):
    n = hfeats.shape[0]
    dc = cfeats.shape[1]
    dcp = 8
    h_dim = W_ne.shape[1]

    src = edge_index[0].astype(jnp.int32)
    dst = edge_index[1].astype(jnp.int32)

    cf_pad = jnp.pad(cfeats, ((0, 0), (0, dcp - dc)))
    W_ce_pad = jnp.pad(W_ce, ((0, dcp - dc), (0, 0)))
    W_c1_pad = jnp.pad(W_c1, ((0, 0), (0, dcp - dc)))
    b_c1_pad = jnp.pad(b_c1, (0, dcp - dc))

    perm = jnp.asarray(_interleave_perm(h_dim, 16))
    W_ne_p = W_ne[:, perm]
    b_ne_p = b_ne[perm]
    W_ce_pad_p = W_ce_pad[:, perm]
    b_ce_p = b_ce[perm]

    hc_bf, c = _node_prep(hfeats, cf_pad, W_ne_p, b_ne_p, W_ce_pad,
                          b_ce, W_ce_pad_p, b_ce_p, n_blk=1000)
    hc_packed = lax.bitcast_convert_type(
        hc_bf.reshape(n, h_dim, 2), jnp.float32)
    ef_t = efeats.T
    e_edges = efeats.shape[0]
    half = e_edges // 2
    n_pad = 10240
    e1, efo_t1 = _edge_prep(ef_t, W_ee, b_ee, W_e1, b_e1,
                            e_blk=2560, off_blk=0, n_blk=125)
    k_a = _sc_edge_kernel(src[:half], dst[:half], e1, hc_packed, n_pad)
    e2, efo_t2 = _edge_prep(ef_t, W_ee, b_ee, W_e1, b_e1,
                            e_blk=2560, off_blk=125, n_blk=125)
    k_b = _sc_edge_kernel(src[half:], dst[half:], e2, hc_packed, n_pad)
    efeats_out = jnp.concatenate([efo_t1, efo_t2], axis=1).T

    hfeats_out, cf_out_pad = _node_post(
        k_a[0, :n], k_a[1, :n], k_b[0, :n], k_b[1, :n], hfeats, cf_pad, c,
        W_n1, b_n1, W_n2, b_n2, W_c1_pad, b_c1_pad, n_blk=1000)
    cfeats_out = cf_out_pad[:, :dc]
    return (hfeats_out, cfeats_out, efeats_out)
